# baseline (speedup 1.0000x reference)
import functools

import jax
import jax.numpy as jnp
from jax import lax
from jax.experimental import pallas as pl
from jax.experimental.pallas import tpu as pltpu
from jax.experimental.pallas import tpu_sc as plsc

EMBED_DIM = 64
BATCH = 16384
HIST = 50
VOCAB = 1000000
TOTAL = BATCH * HIST

NUM_CORES = 2
NUM_SUBCORES = 16
NW = NUM_CORES * NUM_SUBCORES
COLS = BATCH // NW
NBUF = 2
SUBBLK = 128
NSUB = COLS // SUBBLK


def _build_gather():
    mesh = plsc.VectorSubcoreMesh(core_axis_name="c", subcore_axis_name="s")

    @functools.partial(
        pl.kernel,
        mesh=mesh,
        out_type=jax.ShapeDtypeStruct((HIST, 8, BATCH // 128, 8, 128),
                                      jnp.float32),
        scratch_types=[
            pltpu.VMEM((HIST, COLS), jnp.int32),
            pltpu.VMEM((COLS, EMBED_DIM), jnp.float32),
            pltpu.VMEM((COLS, EMBED_DIM), jnp.float32),
            pltpu.VMEM((2, 8, 8, 128), jnp.float32),
            pltpu.SemaphoreType.DMA,
            pltpu.SemaphoreType.DMA,
            pltpu.SemaphoreType.DMA,
            pltpu.SemaphoreType.DMA,
        ],
        compiler_params=pltpu.CompilerParams(use_tc_tiling_on_sc=False,
                                             needs_layout_passes=False),
    )
    def gather_kernel(idx_hbm, table_hbm, out_hbm,
                      idx_all, rows0, rows1, stage, sg0, sg1, st0, st1):
        wid = lax.axis_index("s") * NUM_CORES + lax.axis_index("c")
        col0 = wid * COLS
        blk0 = wid * NSUB

        rows = (rows0, rows1)
        sg = (sg0, sg1)
        st = (st0, st1)

        lane = lax.iota(jnp.int32, 16)

        pltpu.sync_copy(idx_hbm.at[:, pl.ds(col0, COLS)], idx_all)

        pltpu.async_copy(table_hbm.at[idx_all.at[0]], rows0, sg0)
        pltpu.async_copy(table_hbm.at[idx_all.at[1]], rows1, sg1)

        def do_h(h, b):
            pltpu.make_async_copy(table_hbm.at[idx_all.at[h]],
                                  rows[b], sg[b]).wait()
            for s in range(NSUB):
                b2 = s % 2
                if s >= 2:
                    pltpu.make_async_copy(
                        stage.at[b2], out_hbm.at[h, :, 0], st[b2]).wait()
                else:
                    @pl.when(h > 0)
                    def _():
                        pltpu.make_async_copy(
                            stage.at[b2], out_hbm.at[h, :, 0], st[b2]).wait()

                def do_d(d, carry):
                    dg = d >> 3
                    dr = d & 7
                    col = lax.broadcast(d, (16,))
                    for m in range(SUBBLK // 16):
                        row_idx = lane + (SUBBLK * s + 16 * m)
                        vals = plsc.load_gather(rows[b], [row_idx, col])
                        stage[b2, dg, dr, pl.ds(16 * m, 16)] = vals
                    return carry

                lax.fori_loop(0, EMBED_DIM, do_d, 0)

                pltpu.async_copy(stage.at[b2],
                                 out_hbm.at[h, :, blk0 + s], st[b2])

            @pl.when(h + NBUF < HIST)
            def _():
                pltpu.async_copy(table_hbm.at[idx_all.at[h + NBUF]],
                                 rows[b], sg[b])

        def outer(i, carry):
            for b in range(NBUF):
                do_h(NBUF * i + b, b)
            return carry

        lax.fori_loop(0, HIST // NBUF, outer, 0)

        pltpu.make_async_copy(stage.at[0], out_hbm.at[0, :, 0], st[0]).wait()
        pltpu.make_async_copy(stage.at[1], out_hbm.at[0, :, 0], st[1]).wait()

    return gather_kernel


_gather = _build_gather()


@jax.jit
def kernel(data, table):
    idx_hm = data.T.astype(jnp.int32)
    o5 = _gather(idx_hm, table)
    return o5.transpose(2, 4, 0, 1, 3).reshape(BATCH, HIST, EMBED_DIM)


# speedup vs baseline: 1.5229x; 1.5229x over previous
import functools

import jax
import jax.numpy as jnp
from jax import lax
from jax.experimental import pallas as pl
from jax.experimental.pallas import tpu as pltpu
from jax.experimental.pallas import tpu_sc as plsc

EMBED_DIM = 64
BATCH = 16384
HIST = 50
VOCAB = 1000000
TOTAL = BATCH * HIST

NUM_CORES = 2
NUM_SUBCORES = 16
NW = NUM_CORES * NUM_SUBCORES
COLS = BATCH // NW
CBLK = COLS // 128
NBUF = 2


def _detile_body(x_ref, o_ref):
    o_ref[...] = x_ref[...].reshape(HIST * BATCH // 128, 128)


_detile = pl.pallas_call(
    _detile_body,
    out_shape=jax.ShapeDtypeStruct((HIST * BATCH // 128, 128), jnp.int32),
)


def _build_gather():
    mesh = plsc.VectorSubcoreMesh(core_axis_name="c", subcore_axis_name="s")

    @functools.partial(
        pl.kernel,
        mesh=mesh,
        out_type=jax.ShapeDtypeStruct((TOTAL, EMBED_DIM), jnp.float32),
        scratch_types=[
            pltpu.VMEM((HIST, CBLK, 128), jnp.int32),
            pltpu.VMEM((COLS, EMBED_DIM), jnp.float32),
            pltpu.VMEM((COLS, EMBED_DIM), jnp.float32),
            pltpu.SemaphoreType.DMA,
            pltpu.SemaphoreType.DMA,
            pltpu.SemaphoreType.DMA,
            pltpu.SemaphoreType.DMA,
        ],
        compiler_params=pltpu.CompilerParams(use_tc_tiling_on_sc=False),
    )
    def gather_kernel(idx_hbm, table_hbm, out_hbm,
                      idx_all, rows0, rows1, sg0, sg1, so0, so1):
        wid = lax.axis_index("s") * NUM_CORES + lax.axis_index("c")
        col0 = wid * COLS

        rows = (rows0, rows1)
        sg = (sg0, sg1)
        so = (so0, so1)

        def start_gather(h, b):
            for j in range(CBLK):
                pltpu.async_copy(table_hbm.at[idx_all.at[h, j]],
                                 rows[b].at[pl.ds(128 * j, 128)], sg[b])

        pltpu.sync_copy(idx_hbm.at[:, pl.ds(wid * CBLK, CBLK), :], idx_all)

        start_gather(0, 0)
        start_gather(1, 1)

        def outer(i, carry):
            for b in range(NBUF):
                h = NBUF * i + b
                pltpu.make_async_copy(
                    table_hbm.at[idx_all.at[h, 0]], rows[b], sg[b]).wait()
                out_dma = pltpu.async_copy(
                    rows[b],
                    out_hbm.at[pl.ds(h * BATCH + col0, COLS)], so[b])
                out_dma.wait()

                @pl.when(h + NBUF < HIST)
                def _():
                    start_gather(h + NBUF, b)
            return carry

        lax.fori_loop(0, HIST // NBUF, outer, 0)

    return gather_kernel


_gather = _build_gather()


@jax.jit
def kernel(data, table):
    idx_lin = _detile(data.T.astype(jnp.int32))
    flat = _gather(idx_lin.reshape(HIST, BATCH // 128, 128), table)
    return flat.reshape(HIST, BATCH, EMBED_DIM).transpose(1, 0, 2)
